# baseline (speedup 1.0000x reference)
import functools

import jax
import jax.numpy as jnp
from jax import lax
from jax.experimental import pallas as pl
from jax.experimental.pallas import tpu as pltpu
from jax.experimental.pallas import tpu_sc as plsc

N = 65536
C = 20
M = 256
J = 16
KP = C * J
KT = KP + J

NC = 2
NS = 16
NW = NC * NS
NPW = N // NW
L = 16
NG = NPW // L


def _table_body(lbt_ref, lpi_ref, tab_ref):
    lam = lbt_ref[:]
    mx = jnp.max(lam, axis=2, keepdims=True)
    e = jnp.exp(lam - mx)
    B = e / jnp.sum(e, axis=2, keepdims=True)
    lpi = lpi_ref[:]
    pmx = jnp.max(lpi, axis=0, keepdims=True)
    pe = jnp.exp(lpi - pmx)
    Pi = pe / jnp.sum(pe, axis=0, keepdims=True)
    T = B * Pi[:, :, None]
    denom = jnp.sum(T, axis=0)
    tab_ref[0:KP, :] = (T / denom[None, :, :]).reshape(KP, M)
    tab_ref[KP:KT, :] = jnp.log(denom)


_table = pl.pallas_call(
    _table_body,
    out_shape=jax.ShapeDtypeStruct((KT, M), jnp.float32),
)


def _expand_body(tab_hbm, x_hbm, outp_hbm, outl_hbm,
                 tab_v, x_v, obuf, sems):
    wid = lax.axis_index("s") * NC + lax.axis_index("c")
    nbase = wid * NPW
    ld_t = pltpu.async_copy(tab_hbm, tab_v, sems[0])
    ld_x = pltpu.async_copy(x_hbm.at[pl.ds(nbase, NPW)], x_v, sems[1])
    ld_t.wait()
    ld_x.wait()

    def fill(kbase, b):
        tab8 = tab_v.at[pl.ds(kbase * M, 8 * M)]
        ob = obuf.at[b]

        @plsc.parallel_loop(0, NG, 1, unroll=8)
        def inner(g):
            off = g * L
            xv = x_v[pl.ds(off, L)]
            for jj in range(8):
                ob[jj, pl.ds(off, L)] = plsc.load_gather(tab8, [xv + jj * M])

    dsts = [outp_hbm.at[grp // 2, pl.ds((grp % 2) * 8, 8), pl.ds(nbase, NPW)]
            for grp in range(KP // 8)]
    dsts += [outl_hbm.at[pl.ds(j8 * 8, 8), pl.ds(nbase, NPW)]
             for j8 in range(2)]

    pending = [None, None]
    for i, dst in enumerate(dsts):
        b = i % 2
        if pending[b] is not None:
            pending[b].wait()
        fill(i * 8, b)
        pending[b] = pltpu.async_copy(obuf.at[b], dst, sems[b])
    for p in pending:
        p.wait()


@functools.cache
def _expand():
    return pl.kernel(
        _expand_body,
        mesh=plsc.VectorSubcoreMesh(
            core_axis_name="c", subcore_axis_name="s",
            num_cores=NC, num_subcores=NS),
        out_type=[
            jax.ShapeDtypeStruct((C, J, N), jnp.float32),
            jax.ShapeDtypeStruct((J, N), jnp.float32),
        ],
        scratch_types=[
            pltpu.VMEM((KT * M,), jnp.float32),
            pltpu.VMEM((NPW,), jnp.int32),
            pltpu.VMEM((2, 8, NPW), jnp.float32),
            [pltpu.SemaphoreType.DMA, pltpu.SemaphoreType.DMA],
        ],
        compiler_params=pltpu.CompilerParams(
            use_tc_tiling_on_sc=True, needs_layout_passes=False),
    )


def kernel(x, lambda_B, lambda_Pi):
    lbt = jnp.transpose(lambda_B, (0, 2, 1))
    tab = _table(lbt, lambda_Pi).reshape(KT * M)
    xi = x.astype(jnp.int32)
    outp, outl = _expand()(tab, xi)
    return (jnp.transpose(outl, (1, 0)), jnp.transpose(outp, (2, 0, 1)))


# speedup vs baseline: 1.1253x; 1.1253x over previous
import functools

import jax
import jax.numpy as jnp
from jax import lax
from jax.experimental import pallas as pl
from jax.experimental.pallas import tpu as pltpu
from jax.experimental.pallas import tpu_sc as plsc

N = 65536
C = 20
M = 256
J = 16
KP = C * J
KT = KP + J

NC = 2
NS = 16
NW = NC * NS
L = 16
KH = KP // 2
NPW2 = N // (NW // 2)
NG2 = NPW2 // L


def _table_body(lbt_ref, lpi_ref, tab_ref):
    lam = lbt_ref[:]
    mx = jnp.max(lam, axis=2, keepdims=True)
    e = jnp.exp(lam - mx)
    B = e / jnp.sum(e, axis=2, keepdims=True)
    lpi = lpi_ref[:]
    pmx = jnp.max(lpi, axis=0, keepdims=True)
    pe = jnp.exp(lpi - pmx)
    Pi = pe / jnp.sum(pe, axis=0, keepdims=True)
    T = B * Pi[:, :, None]
    denom = jnp.sum(T, axis=0)
    tab_ref[0:KP, :] = (T / denom[None, :, :]).reshape(KP, M)
    tab_ref[KP:KT, :] = jnp.log(denom)


_table = pl.pallas_call(
    _table_body,
    out_shape=jax.ShapeDtypeStruct((KT, M), jnp.float32),
)


def _expand_body(tab_hbm, x_hbm, outp_hbm, outl_hbm,
                 tab_v, x_v, obuf, sems):
    wid = lax.axis_index("s") * NC + lax.axis_index("c")
    kh = wid % 2
    nbase = (wid // 2) * NPW2
    ld_p = pltpu.async_copy(
        tab_hbm.at[pl.ds(kh * (KH * M), KH * M)], tab_v.at[pl.ds(0, KH * M)],
        sems[0])
    ld_l = pltpu.async_copy(
        tab_hbm.at[pl.ds(KP * M + kh * (8 * M), 8 * M)],
        tab_v.at[pl.ds(KH * M, 8 * M)], sems[1])
    ld_x = pltpu.async_copy(x_hbm.at[pl.ds(nbase, NPW2)], x_v, sems[0])
    ld_p.wait()
    ld_l.wait()
    ld_x.wait()

    def fill(kbase, b):
        tab8 = tab_v.at[pl.ds(kbase * M, 8 * M)]
        ob = obuf.at[b]

        @plsc.parallel_loop(0, NG2, 1, unroll=8)
        def inner(g):
            off = g * L
            xv = x_v[pl.ds(off, L)]
            for jj in range(8):
                ob[jj, pl.ds(off, L)] = plsc.load_gather(tab8, [xv + jj * M])

    dsts = [outp_hbm.at[kh * (KH // J) + grp // 2, pl.ds((grp % 2) * 8, 8),
                        pl.ds(nbase, NPW2)]
            for grp in range(KH // 8)]
    dsts += [outl_hbm.at[pl.ds(kh * 8, 8), pl.ds(nbase, NPW2)]]

    pending = [None, None]
    for i, dst in enumerate(dsts):
        b = i % 2
        if pending[b] is not None:
            pending[b].wait()
        fill(i * 8, b)
        pending[b] = pltpu.async_copy(obuf.at[b], dst, sems[b])
    for p in pending:
        p.wait()


@functools.cache
def _expand():
    return pl.kernel(
        _expand_body,
        mesh=plsc.VectorSubcoreMesh(
            core_axis_name="c", subcore_axis_name="s",
            num_cores=NC, num_subcores=NS),
        out_type=[
            jax.ShapeDtypeStruct((C, J, N), jnp.float32),
            jax.ShapeDtypeStruct((J, N), jnp.float32),
        ],
        scratch_types=[
            pltpu.VMEM(((KH + 8) * M,), jnp.float32),
            pltpu.VMEM((NPW2,), jnp.int32),
            pltpu.VMEM((2, 8, NPW2), jnp.float32),
            [pltpu.SemaphoreType.DMA, pltpu.SemaphoreType.DMA],
        ],
        compiler_params=pltpu.CompilerParams(
            use_tc_tiling_on_sc=True, needs_layout_passes=False),
    )


def kernel(x, lambda_B, lambda_Pi):
    lbt = jnp.transpose(lambda_B, (0, 2, 1))
    tab = _table(lbt, lambda_Pi).reshape(KT * M)
    xi = x.astype(jnp.int32)
    outp, outl = _expand()(tab, xi)
    return (jnp.transpose(outl, (1, 0)), jnp.transpose(outp, (2, 0, 1)))
